# Initial kernel scaffold; baseline (speedup 1.0000x reference)
import jax
import jax.numpy as jnp
from jax.experimental import pallas as pl


def kernel(inputs):
    raise NotImplementedError("write your pallas kernel here")


# speedup vs baseline: 40.1625x; 40.1625x over previous
import functools

import jax
import jax.numpy as jnp
from jax import lax
from jax.experimental import pallas as pl
from jax.experimental.pallas import tpu as pltpu
from jax.experimental.pallas import tpu_sc as plsc

B, S, C = 32, 32768, 64
K = 8
L = 16
G = C // L
R = 512
NCHUNK = S // R


def _insert(acc, v):
    out = []
    new = v
    for a in acc:
        hi = jnp.maximum(a, new)
        new = jnp.minimum(a, new)
        out.append(hi)
    return out


def _sc_topk(x):
    mesh = plsc.VectorSubcoreMesh(core_axis_name="c", subcore_axis_name="s")

    @functools.partial(
        pl.kernel,
        mesh=mesh,
        out_type=jax.ShapeDtypeStruct((B, K, C), jnp.float32),
        scratch_types=[
            pltpu.VMEM((R, C), jnp.float32),
            pltpu.VMEM((K, C), jnp.float32),
            pltpu.SemaphoreType.DMA,
        ],
    )
    def run(x_hbm, out_hbm, buf, outv, sem):
        wid = lax.axis_index("s") * 2 + lax.axis_index("c")
        neg = jnp.full((L,), -jnp.inf, jnp.float32)
        accs0 = tuple([neg] * (K * G))

        def chunk_body(c, accs):
            pltpu.async_copy(x_hbm.at[wid, pl.ds(c * R, R)], buf, sem).wait()

            def row_body(r, accs):
                accs = list(accs)
                for g in range(G):
                    v = buf[r, pl.ds(L * g, L)]
                    accs[g * K:(g + 1) * K] = _insert(accs[g * K:(g + 1) * K], v)
                return tuple(accs)

            return lax.fori_loop(0, R, row_body, accs)

        accs = lax.fori_loop(0, NCHUNK, chunk_body, accs0)
        for i in range(K):
            for g in range(G):
                outv[i, pl.ds(L * g, L)] = accs[g * K + i]
        pltpu.sync_copy(outv, out_hbm.at[wid])

    return run(x)


def kernel(inputs):
    return _sc_topk(inputs)


# speedup vs baseline: 46.1067x; 1.1480x over previous
import functools

import jax
import jax.numpy as jnp
from jax import lax
from jax.experimental import pallas as pl
from jax.experimental.pallas import tpu as pltpu
from jax.experimental.pallas import tpu_sc as plsc

B, S, C = 32, 32768, 64
K = 8
L = 16
G = C // L
RB = 8
R = 128
NBUF = 4
NCHUNK = S // R
NBLK = R // RB

_SORT8 = [(0, 1), (2, 3), (4, 5), (6, 7), (0, 2), (1, 3), (4, 6), (5, 7),
          (1, 2), (5, 6), (0, 4), (1, 5), (2, 6), (3, 7), (2, 4), (3, 5),
          (1, 2), (3, 4), (5, 6)]


def _sort8(v):
    v = list(v)
    for i, j in _SORT8:
        hi = jnp.maximum(v[i], v[j])
        lo = jnp.minimum(v[i], v[j])
        v[i], v[j] = hi, lo
    return v


def _merge_topk(acc, srt):
    c = [jnp.maximum(acc[i], srt[K - 1 - i]) for i in range(K)]
    for d in (4, 2, 1):
        for i in range(K):
            if i & d == 0:
                hi = jnp.maximum(c[i], c[i + d])
                lo = jnp.minimum(c[i], c[i + d])
                c[i], c[i + d] = hi, lo
    return c


def _sc_topk(x):
    mesh = plsc.VectorSubcoreMesh(core_axis_name="c", subcore_axis_name="s")

    @functools.partial(
        pl.kernel,
        mesh=mesh,
        compiler_params=pltpu.CompilerParams(needs_layout_passes=False),
        out_type=jax.ShapeDtypeStruct((B, K, C), jnp.float32),
        scratch_types=[
            pltpu.VMEM((NBUF, R, C), jnp.float32),
            pltpu.VMEM((K, C), jnp.float32),
            pltpu.SemaphoreType.DMA,
            pltpu.SemaphoreType.DMA,
            pltpu.SemaphoreType.DMA,
            pltpu.SemaphoreType.DMA,
        ],
    )
    def run(x_hbm, out_hbm, buf, outv, *sems):
        wid = lax.axis_index("s") * 2 + lax.axis_index("c")
        neg = jnp.full((L,), -jnp.inf, jnp.float32)
        accs0 = tuple([neg] * (K * G))

        def start(chunk, b):
            pltpu.make_async_copy(
                x_hbm.at[wid, pl.ds(chunk * R, R)], buf.at[b], sems[b]
            ).start()

        def wait(b):
            pltpu.make_async_copy(
                x_hbm.at[wid, pl.ds(0, R)], buf.at[b], sems[b]
            ).wait()

        def process(b, accs):
            def blk_body(blk, accs):
                accs = list(accs)
                r0 = blk * RB
                for g in range(G):
                    rows = [buf[b, r0 + j, pl.ds(L * g, L)] for j in range(RB)]
                    m = rows[0]
                    for j in range(1, RB):
                        m = jnp.maximum(m, rows[j])
                    acc_g = accs[g * K:(g + 1) * K]
                    need = jnp.any(m > acc_g[K - 1])

                    def ins(rows=rows, acc_g=acc_g):
                        return tuple(_merge_topk(acc_g, _sort8(rows)))

                    def keep(acc_g=acc_g):
                        return tuple(acc_g)

                    accs[g * K:(g + 1) * K] = lax.cond(need, ins, keep)
                return tuple(accs)

            return lax.fori_loop(0, NBLK, blk_body, accs)

        for b in range(NBUF):
            start(b, b)

        def ring_body(q, accs):
            for b in range(NBUF):
                wait(b)
                accs = process(b, accs)
                refill = jnp.minimum(q * NBUF + b + NBUF, NCHUNK - 1)
                start(refill, b)
            return accs

        accs = lax.fori_loop(0, NCHUNK // NBUF - 1, ring_body, accs0)
        for b in range(NBUF):
            wait(b)
            accs = process(b, accs)
            start(NCHUNK - 1, b)
        for b in range(NBUF):
            wait(b)

        for i in range(K):
            for g in range(G):
                outv[i, pl.ds(L * g, L)] = accs[g * K + i]
        pltpu.sync_copy(outv, out_hbm.at[wid])

    return run(x)


def kernel(inputs):
    return _sc_topk(inputs)


# speedup vs baseline: 56.1114x; 1.2170x over previous
import functools

import jax
import jax.numpy as jnp
from jax import lax
from jax.experimental import pallas as pl
from jax.experimental.pallas import tpu as pltpu
from jax.experimental.pallas import tpu_sc as plsc

B, S, C = 32, 32768, 64
K = 8
L = 16
G = C // L
SB = 64
NSUB = SB // K
R = 256
NBUF = 2
NCHUNK = S // R
NSBCHUNK = R // SB

_SORT8 = [(0, 1), (2, 3), (4, 5), (6, 7), (0, 2), (1, 3), (4, 6), (5, 7),
          (1, 2), (5, 6), (0, 4), (1, 5), (2, 6), (3, 7), (2, 4), (3, 5),
          (1, 2), (3, 4), (5, 6)]


def _sort8(v):
    v = list(v)
    for i, j in _SORT8:
        hi = jnp.maximum(v[i], v[j])
        lo = jnp.minimum(v[i], v[j])
        v[i], v[j] = hi, lo
    return v


def _merge_topk(acc, srt):
    c = [jnp.maximum(acc[i], srt[K - 1 - i]) for i in range(K)]
    for d in (4, 2, 1):
        for i in range(K):
            if i & d == 0:
                hi = jnp.maximum(c[i], c[i + d])
                lo = jnp.minimum(c[i], c[i + d])
                c[i], c[i + d] = hi, lo
    return c


def _sc_topk(x):
    mesh = plsc.VectorSubcoreMesh(core_axis_name="c", subcore_axis_name="s")

    @functools.partial(
        pl.kernel,
        mesh=mesh,
        compiler_params=pltpu.CompilerParams(needs_layout_passes=False),
        out_type=jax.ShapeDtypeStruct((B, K, C), jnp.float32),
        scratch_types=[
            pltpu.VMEM((NBUF, R, C), jnp.float32),
            pltpu.VMEM((K, C), jnp.float32),
            pltpu.SemaphoreType.DMA,
            pltpu.SemaphoreType.DMA,
        ],
    )
    def run(x_hbm, out_hbm, buf, acc, *sems):
        wid = lax.axis_index("s") * 2 + lax.axis_index("c")
        neg = jnp.full((L,), -jnp.inf, jnp.float32)

        def start(chunk, b):
            pltpu.make_async_copy(
                x_hbm.at[wid, pl.ds(chunk * R, R)], buf.at[b], sems[b]
            ).start()

        def wait(b):
            pltpu.make_async_copy(
                x_hbm.at[wid, pl.ds(0, R)], buf.at[b], sems[b]
            ).wait()

        for b in range(NBUF):
            start(b, b)
        for i in range(K):
            for g in range(G):
                acc[i, pl.ds(L * g, L)] = neg

        def process(b):
            def sb_body(sb, carry):
                r0 = sb * SB
                for g in range(G):
                    sl = pl.ds(L * g, L)
                    thr = acc[K - 1, sl]
                    bits = jnp.zeros((L,), jnp.int32)
                    for k in range(NSUB):
                        m = buf[b, r0 + k * K, sl]
                        for j in range(1, K):
                            m = jnp.maximum(m, buf[b, r0 + k * K + j, sl])
                        cnt = plsc.all_reduce_population_count(m > thr)
                        bits = bits | (jnp.minimum(cnt, 1) << k)
                    bmap = jnp.max(bits)

                    def ins_body(k2, carry2, g=g, sl=sl, r0=r0, bmap=bmap):
                        @pl.when(((bmap >> k2) & 1) == 1)
                        def _():
                            base = r0 + k2 * K
                            rows = [buf[b, base + j, sl] for j in range(K)]
                            acc_g = [acc[i, sl] for i in range(K)]
                            new = _merge_topk(acc_g, _sort8(rows))
                            for i in range(K):
                                acc[i, sl] = new[i]
                        return carry2

                    lax.fori_loop(0, NSUB, ins_body, 0)
                return carry

            lax.fori_loop(0, NSBCHUNK, sb_body, 0)

        def ring_body(q, carry):
            for b in range(NBUF):
                wait(b)
                process(b)
                refill = jnp.minimum(q * NBUF + b + NBUF, NCHUNK - 1)
                start(refill, b)
            return carry

        lax.fori_loop(0, NCHUNK // NBUF - 1, ring_body, 0)
        for b in range(NBUF):
            wait(b)
            process(b)
            start(NCHUNK - 1, b)
        for b in range(NBUF):
            wait(b)

        pltpu.sync_copy(acc, out_hbm.at[wid])

    return run(x)


def kernel(inputs):
    return _sc_topk(inputs)
